# baseline (speedup 1.0000x reference)
import functools

import jax
import jax.numpy as jnp
from jax import lax
from jax.experimental import pallas as pl
from jax.experimental.pallas import tpu as pltpu
from jax.experimental.pallas import tpu_sc as plsc

N = 10000
E = 320000
D = 128
H = 128
T = 3
NUM_BASES = 8
SH_DIM = 9
BOND_DIM = 16
R_MAX = 5.0

NC = 2
NS = 16
NW = NC * NS

CH = 80
KBUF = 5
EPW = E // NW
NPAD = 10240

_mesh = plsc.VectorSubcoreMesh(core_axis_name="c", subcore_axis_name="s",
                               num_cores=NC, num_subcores=NS)


def _wid():
    return lax.axis_index("c") * NS + lax.axis_index("s")


def _gather_small(tbl_hbm, idx_v, out_hbm, rows, sems, bpw, base):
    nchunks = bpw // CH
    descs = []
    for c in range(nchunks):
        descs.append(pltpu.async_copy(
            tbl_hbm.at[idx_v.at[pl.ds(c * CH, CH)]], rows[c], sems[c]))
    for c in range(nchunks):
        descs[c].wait()
        pltpu.sync_copy(rows[c], out_hbm.at[pl.ds(base + c * CH, CH), :])


GCH = 128
GK = 6
_GNFULL = EPW // GCH
_GTAIL = EPW - _GNFULL * GCH


def _gather_job(tbl_hbm, idx_v, out_hbm, rows, gsems, osems, base):
    def _wait_out(b):
        pltpu.make_async_copy(
            rows[b], out_hbm.at[pl.ds(base, GCH), :], osems[b]).wait()

    @pl.loop(0, _GNFULL // GK)
    def _group(g):
        c0 = g * GK
        descs = []
        for b in range(GK):
            @pl.when(g > 0)
            def _w(b=b):
                _wait_out(b)
            descs.append(pltpu.async_copy(
                tbl_hbm.at[idx_v.at[pl.ds((c0 + b) * GCH, GCH)]],
                rows[b], gsems[b]))
        for b in range(GK):
            descs[b].wait()
            pltpu.async_copy(
                rows[b], out_hbm.at[pl.ds(base + (c0 + b) * GCH, GCH), :],
                osems[b])

    for b in range(GK):
        _wait_out(b)
    toff = _GNFULL * GCH
    pltpu.async_copy(
        tbl_hbm.at[idx_v.at[pl.ds(toff, _GTAIL)]],
        rows[0].at[pl.ds(0, _GTAIL), :], gsems[0]).wait()
    pltpu.sync_copy(rows[0].at[pl.ds(0, _GTAIL), :],
                    out_hbm.at[pl.ds(base + toff, _GTAIL), :])


def _sc_gatherpos_body(pos_hbm, send_hbm, recv_hbm,
                       ps_hbm, pr_hbm, idx_v, *bufs):
    wid = _wid()
    base = wid * EPW
    rows = list(bufs[:GK])
    gsems = list(bufs[GK:2 * GK])
    osems = list(bufs[2 * GK:3 * GK])
    for idx_hbm, tbl, out in ((send_hbm, pos_hbm, ps_hbm),
                              (recv_hbm, pos_hbm, pr_hbm)):
        pltpu.sync_copy(idx_hbm.at[pl.ds(base, EPW)], idx_v)
        _gather_job(tbl, idx_v, out, rows, gsems, osems, base)


_sc_gatherpos = pl.kernel(
    _sc_gatherpos_body,
    out_type=[jax.ShapeDtypeStruct((E, 16), jnp.float32),
              jax.ShapeDtypeStruct((E, 16), jnp.float32)],
    mesh=_mesh,
    compiler_params=pltpu.CompilerParams(use_tc_tiling_on_sc=False),
    scratch_types=[pltpu.VMEM((EPW,), jnp.int32)]
    + [pltpu.VMEM((GCH, 16), jnp.float32) for _ in range(GK)]
    + [pltpu.SemaphoreType.DMA for _ in range(2 * GK)],
)


def _sc_gather_atoms_body(tbl_hbm, idx_hbm, out_hbm,
                          idx_v, r0, r1, r2, r3, s0, s1, s2, s3):
    wid = _wid()
    bpw = NPAD // NW
    base = wid * bpw
    pltpu.sync_copy(idx_hbm.at[pl.ds(base, bpw)], idx_v)
    _gather_small(tbl_hbm, idx_v, out_hbm, [r0, r1, r2, r3],
                  [s0, s1, s2, s3], bpw, base)


_sc_gather_atoms = pl.kernel(
    _sc_gather_atoms_body,
    out_type=jax.ShapeDtypeStruct((NPAD, D), jnp.float32),
    mesh=_mesh,
    compiler_params=pltpu.CompilerParams(use_tc_tiling_on_sc=False),
    scratch_types=[pltpu.VMEM((NPAD // NW,), jnp.int32)]
    + [pltpu.VMEM((CH, D), jnp.float32) for _ in range(4)]
    + [pltpu.SemaphoreType.DMA for _ in range(4)],
)


def _sc_gather2_body(ts_hbm, send_hbm, tr_hbm, recv_hbm,
                     as_hbm, ar_hbm, idx_v, *bufs):
    wid = _wid()
    base = wid * EPW
    rows = list(bufs[:GK])
    gsems = list(bufs[GK:2 * GK])
    osems = list(bufs[2 * GK:3 * GK])
    for idx_hbm, tbl, out in ((send_hbm, ts_hbm, as_hbm),
                              (recv_hbm, tr_hbm, ar_hbm)):
        pltpu.sync_copy(idx_hbm.at[pl.ds(base, EPW)], idx_v)
        _gather_job(tbl, idx_v, out, rows, gsems, osems, base)


_sc_gather2 = pl.kernel(
    _sc_gather2_body,
    out_type=[jax.ShapeDtypeStruct((E, 72), jnp.float32),
              jax.ShapeDtypeStruct((E, 72), jnp.float32)],
    mesh=_mesh,
    compiler_params=pltpu.CompilerParams(use_tc_tiling_on_sc=False),
    scratch_types=[pltpu.VMEM((EPW,), jnp.int32)]
    + [pltpu.VMEM((GCH, 72), jnp.float32) for _ in range(GK)]
    + [pltpu.SemaphoreType.DMA for _ in range(2 * GK)],
)


_NCHUNK = EPW // CH
_ROWS_PER_TILE = N // NS


def _sc_scatter_body(m_hbm, recv2d_hbm, zeros_hbm, out_hbm,
                     idxs_v, mb0, mb1, acc_sh, lm0, lm1, ss0, ss1):
    cid = lax.axis_index("c")
    sid = lax.axis_index("s")
    wid = cid * NS + sid
    base = wid * EPW
    pltpu.sync_copy(zeros_hbm.at[pl.ds(sid * _ROWS_PER_TILE, _ROWS_PER_TILE), :],
                    acc_sh.at[pl.ds(sid * _ROWS_PER_TILE, _ROWS_PER_TILE), :])
    pltpu.sync_copy(recv2d_hbm.at[pl.ds(wid * _NCHUNK, _NCHUNK), :], idxs_v)
    plsc.subcore_barrier()

    mbufs = (mb0, mb1)
    lsems = (lm0, lm1)
    ssems = (ss0, ss1)

    @pl.loop(0, _NCHUNK // 2)
    def _group(g):
        loads = []
        for b in range(2):
            loads.append(pltpu.async_copy(
                m_hbm.at[pl.ds(base + (g * 2 + b) * CH, CH), :],
                mbufs[b], lsems[b]))
        scats = []
        for b in range(2):
            loads[b].wait()
            scats.append(pltpu.async_copy(
                mbufs[b], acc_sh.at[idxs_v.at[g * 2 + b]],
                ssems[b], add=True))
        for b in range(2):
            scats[b].wait()

    c = _NCHUNK - 1
    pltpu.async_copy(m_hbm.at[pl.ds(base + c * CH, CH), :], mb0, lm0).wait()
    pltpu.async_copy(mb0, acc_sh.at[idxs_v.at[c]], ss0, add=True).wait()

    plsc.subcore_barrier()
    pltpu.sync_copy(acc_sh.at[pl.ds(sid * _ROWS_PER_TILE, _ROWS_PER_TILE), :],
                    out_hbm.at[cid, pl.ds(sid * _ROWS_PER_TILE, _ROWS_PER_TILE), :])


_sc_scatter = pl.kernel(
    _sc_scatter_body,
    out_type=jax.ShapeDtypeStruct((NC, N, D), jnp.float32),
    mesh=_mesh,
    compiler_params=pltpu.CompilerParams(use_tc_tiling_on_sc=False),
    scratch_types=[pltpu.VMEM((EPW // CH, CH), jnp.int32),
                   pltpu.VMEM((CH, D), jnp.float32),
                   pltpu.VMEM((CH, D), jnp.float32),
                   pltpu.VMEM_SHARED((N, D), jnp.float32)]
    + [pltpu.SemaphoreType.DMA for _ in range(4)],
)


BE = 2000
BN = 2000

_S3 = 3.0 ** 0.5
_S5 = 5.0 ** 0.5
_S15 = 15.0 ** 0.5
_PI = 3.141592653589793


def _rms(x):
    return x * lax.rsqrt(jnp.mean(x * x, axis=-1, keepdims=True) + 1e-6)


def _silu(x):
    return x * jax.nn.sigmoid(x)


def _geom_body(ps_ref, pr_ref, oh_ref, ones_ref, mxyz_ref, mb8_ref,
               wp_ref, wbond_ref, b0_ref, e0_ref, env_ref):
    f32 = jnp.float32
    vec = ps_ref[:, :] - pr_ref[:, :]
    d2 = jnp.dot(vec * vec, ones_ref[:, :],
                 preferred_element_type=f32) + 1e-12
    inv_d = lax.rsqrt(d2)
    d = d2 * inv_d
    inv_dd = 1.0 / (d + 1e-9)
    xyz = jnp.dot(vec, mxyz_ref[:, :], preferred_element_type=f32)
    xu = xyz[:, 0:16] * inv_d
    yu = xyz[:, 16:32] * inv_d
    zu = xyz[:, 32:48] * inv_d
    lane = lax.broadcasted_iota(jnp.int32, (BE, 16), 1)
    coef = jnp.where(lane < 8, (lane + 1).astype(f32),
                     jnp.where(lane == 8, 1.0, 0.0))
    off = jnp.where(lane == 8, _PI / 2.0, 0.0)
    s = jnp.sin(d * (coef * (_PI / R_MAX)) + off)
    s8 = jnp.dot(s, mb8_ref[:, :], preferred_element_type=f32)
    env = jnp.where(d < R_MAX, 0.5 * (s8 + 1.0), 0.0)
    g = (s * inv_dd) * (((2.0 / R_MAX) ** 0.5)) * env
    sh_rest = [_S3 * xu, _S3 * yu, _S3 * zu,
               _S15 * xu * yu, _S15 * yu * zu,
               (_S5 / 2.0) * (3.0 * zu * zu - 1.0),
               _S15 * xu * zu, (_S15 / 2.0) * (xu * xu - yu * yu)]
    outer = jnp.concatenate(
        [g] + [g * sj for sj in sh_rest], axis=1)
    e0 = (jnp.dot(outer, wp_ref[:, :], preferred_element_type=f32)
          + jnp.dot(oh_ref[:, :], wbond_ref[:, :], preferred_element_type=f32)
          + b0_ref[:, :])
    e0_ref[:, :] = _pack2(e0, 64)
    env_ref[:, :] = env[:, 0:8]


def _full(shape):
    return pl.BlockSpec(shape, lambda i: (0,) * len(shape))


_tc_geom = pl.pallas_call(
    _geom_body,
    grid=(E // BE,),
    in_specs=[pl.BlockSpec((BE, 16), lambda i: (i, 0)),
              pl.BlockSpec((BE, 16), lambda i: (i, 0)),
              pl.BlockSpec((BE, 8), lambda i: (i, 0)),
              _full((16, 16)), _full((16, 48)), _full((16, 16)),
              _full((144, D)), _full((8, D)), _full((1, D))],
    out_specs=[pl.BlockSpec((BE, 64), lambda i: (i, 0)),
               pl.BlockSpec((BE, 8), lambda i: (i, 0))],
    out_shape=[jax.ShapeDtypeStruct((E, 64), jnp.float32),
               jax.ShapeDtypeStruct((E, 8), jnp.float32)],
    compiler_params=pltpu.CompilerParams(
        dimension_semantics=("arbitrary",)),
)


_HIMASK = -65536


def _pack2(x, k=72):
    hi = lax.bitcast_convert_type(x[:, :k], jnp.int32)
    lo = lax.bitcast_convert_type(x[:, k:], jnp.int32)
    hi = (hi + 0x8000) & _HIMASK
    lo = lax.shift_right_logical(lo + 0x8000, 16)
    return lax.bitcast_convert_type(hi | lo, jnp.float32)


def _unpack2(p):
    u = lax.bitcast_convert_type(p, jnp.int32)
    hi = lax.bitcast_convert_type(u & _HIMASK, jnp.float32)
    lo = lax.bitcast_convert_type(lax.shift_left(u, 16), jnp.float32)
    return jnp.concatenate([hi, lo], axis=1)


def _proj_body(h_ref, wsa_ref, wsb_ref, ts_ref, tr_ref):
    h = h_ref[:, :]
    ts_ref[:, :] = _pack2(
        jnp.dot(h, wsa_ref[:, :], preferred_element_type=jnp.float32))
    tr_ref[:, :] = _pack2(
        jnp.dot(h, wsb_ref[:, :], preferred_element_type=jnp.float32))


_tc_proj = pl.pallas_call(
    _proj_body,
    grid=(N // BN,),
    in_specs=[pl.BlockSpec((BN, D), lambda i: (i, 0)),
              _full((D, 144)), _full((D, 144))],
    out_specs=[pl.BlockSpec((BN, 72), lambda i: (i, 0)),
               pl.BlockSpec((BN, 72), lambda i: (i, 0))],
    out_shape=[jax.ShapeDtypeStruct((N, 72), jnp.float32),
               jax.ShapeDtypeStruct((N, 72), jnp.float32)],
    compiler_params=pltpu.CompilerParams(
        dimension_semantics=("arbitrary",)),
)


def _edge_body(as_ref, ar_ref, e_ref, env_ref, onesd_ref, m8_ref,
               w1cg_ref, b1g_ref, w2_ref, b2_ref, enew_ref, m_ref):
    f32 = jnp.float32
    e = _unpack2(e_ref[:, :])
    tmp = (_unpack2(as_ref[:, :]) + _unpack2(ar_ref[:, :])
           + jnp.dot(e, w1cg_ref[:, :], preferred_element_type=f32)
           + b1g_ref[:, :])
    pre = tmp[:, :D]
    msq = jnp.dot(pre * pre, onesd_ref[:, :],
                  preferred_element_type=f32) * (1.0 / D)
    nrm = pre * lax.rsqrt(msq + 1e-6)
    act = _silu(nrm)
    m2 = jnp.dot(act, w2_ref[:, :], preferred_element_type=f32) + b2_ref[:, :]
    gl8 = tmp[:, D:D + 8]
    gev8 = jax.nn.sigmoid(gl8) * env_ref[:, :]
    m = m2 * jnp.dot(gev8, m8_ref[:, :], preferred_element_type=f32)
    enew_ref[:, :] = _pack2(e + m, 64)
    m_ref[:, :] = m


_tc_edge = pl.pallas_call(
    _edge_body,
    grid=(E // BE,),
    in_specs=[pl.BlockSpec((BE, 72), lambda i: (i, 0)),
              pl.BlockSpec((BE, 72), lambda i: (i, 0)),
              pl.BlockSpec((BE, 64), lambda i: (i, 0)),
              pl.BlockSpec((BE, 8), lambda i: (i, 0)),
              _full((D, D)), _full((8, D)),
              _full((D, 144)), _full((1, 144)),
              _full((H, D)), _full((1, D))],
    out_specs=[pl.BlockSpec((BE, 64), lambda i: (i, 0)),
               pl.BlockSpec((BE, D), lambda i: (i, 0))],
    out_shape=[jax.ShapeDtypeStruct((E, 64), jnp.float32),
               jax.ShapeDtypeStruct((E, D), jnp.float32)],
    compiler_params=pltpu.CompilerParams(
        dimension_semantics=("arbitrary",)),
)


def _node_body(h_ref, p0_ref, p1_ref, u1a_ref, u1b_ref, c1_ref,
               u2_ref, c2_ref, hnew_ref):
    h = h_ref[:, :]
    agg = p0_ref[:, :] + p1_ref[:, :]
    pre = (jnp.dot(h, u1a_ref[:, :], preferred_element_type=jnp.float32)
           + jnp.dot(agg, u1b_ref[:, :], preferred_element_type=jnp.float32)
           + c1_ref[:, :])
    hnew_ref[:, :] = h + jnp.dot(_silu(_rms(pre)), u2_ref[:, :],
                                 preferred_element_type=jnp.float32) + c2_ref[:, :]


_tc_node = pl.pallas_call(
    _node_body,
    grid=(N // BN,),
    in_specs=[pl.BlockSpec((BN, D), lambda i: (i, 0)),
              pl.BlockSpec((BN, D), lambda i: (i, 0)),
              pl.BlockSpec((BN, D), lambda i: (i, 0)),
              _full((D, H)), _full((D, H)), _full((1, H)),
              _full((H, D)), _full((1, D))],
    out_specs=pl.BlockSpec((BN, D), lambda i: (i, 0)),
    out_shape=jax.ShapeDtypeStruct((N, D), jnp.float32),
    compiler_params=pltpu.CompilerParams(
        dimension_semantics=("arbitrary",)),
)


def _head_body(h_ref, w1_ref, b1_ref, w2_ref, b2_ref, out_ref):
    pre = jnp.dot(h_ref[:, :], w1_ref[:, :],
                  preferred_element_type=jnp.float32) + b1_ref[:, :]
    out_ref[:, :] = jnp.dot(_silu(_rms(pre)), w2_ref[:, :],
                            preferred_element_type=jnp.float32) + b2_ref[:, :]


_tc_head = pl.pallas_call(
    _head_body,
    grid=(N // BN,),
    in_specs=[pl.BlockSpec((BN, D), lambda i: (i, 0)),
              _full((D, H)), _full((1, H)),
              _full((H, D)), _full((1, D))],
    out_specs=pl.BlockSpec((BN, D), lambda i: (i, 0)),
    out_shape=jax.ShapeDtypeStruct((N, D), jnp.float32),
    compiler_params=pltpu.CompilerParams(
        dimension_semantics=("arbitrary",)),
)


def kernel(pos, atomic_numbers, senders, receivers, bond_types, batch,
           num_graphs, c_noise, c_in, atom_table, bond_table, W_e0, b_e0,
           W1, b1, W2, b2, Wg, bg, U1, c1, U2, c2, Wh1, bh1, Wh2, bh2):
    del batch, num_graphs, c_noise
    f32 = jnp.float32
    i32 = jnp.int32
    senders = senders.astype(i32)
    receivers = receivers.astype(i32)
    bond_types = bond_types.astype(i32)
    atomic_numbers = atomic_numbers.astype(i32)

    unscaled = pos.astype(f32) / c_in.astype(f32)
    pos_pad = jnp.zeros((N, 16), f32).at[:, :3].set(unscaled)
    at_pad = jnp.zeros((NPAD,), i32).at[:N].set(atomic_numbers)
    recv2d = receivers.reshape(E // CH, CH)
    zeros_nd = jnp.zeros((N, D), f32)

    Wp = W_e0[:NUM_BASES * SH_DIM].reshape(NUM_BASES, SH_DIM, D)
    Wp = Wp.transpose(1, 0, 2)
    Wp = jnp.concatenate(
        [Wp, jnp.zeros((SH_DIM, 16 - NUM_BASES, D), f32)], axis=1
    ).reshape(SH_DIM * 16, D)
    Wbond = jnp.zeros((8, BOND_DIM), f32).at[:5].set(
        bond_table.astype(f32)) @ W_e0[NUM_BASES * SH_DIM:]
    onehot = (bond_types[:, None] == jnp.arange(8, dtype=i32)[None, :]
              ).astype(f32)
    b_e0r = b_e0.reshape(1, D)

    ones16 = jnp.ones((16, 16), f32)
    eye16 = jnp.eye(16, dtype=f32)
    mxyz = jnp.concatenate(
        [jnp.outer(eye16[0], jnp.ones(16, f32)),
         jnp.outer(eye16[1], jnp.ones(16, f32)),
         jnp.outer(eye16[2], jnp.ones(16, f32))], axis=1)
    mb8 = jnp.outer(eye16[8], jnp.ones(16, f32))
    onesd = jnp.ones((D, D), f32)
    m8 = jnp.outer(jnp.eye(8, dtype=f32)[0], jnp.ones(D, f32))

    ps, pr = _sc_gatherpos(pos_pad, senders, receivers)
    h_full = _sc_gather_atoms(atom_table.astype(f32), at_pad)
    h = h_full[:N]

    e, env = _tc_geom(ps, pr, onehot, ones16, mxyz, mb8, Wp, Wbond, b_e0r)

    for t in range(T):
        W1t = W1[t]
        Wgt = Wg[t]
        wsa = jnp.concatenate(
            [W1t[:D], jnp.zeros((D, 16), f32).at[:, 0:1].set(Wgt[:D])], axis=1)
        wsb = jnp.concatenate(
            [W1t[D:2 * D], jnp.zeros((D, 16), f32).at[:, 0:1].set(Wgt[D:2 * D])],
            axis=1)
        w1cg = jnp.concatenate(
            [W1t[2 * D:], jnp.zeros((D, 16), f32).at[:, 0:1].set(Wgt[2 * D:])],
            axis=1)
        b1g = jnp.zeros((1, 144), f32).at[0, :D].set(b1[t]).at[0, D].set(bg[t, 0])

        ts, tr = _tc_proj(h, wsa, wsb)
        a_s, a_r = _sc_gather2(ts, senders, tr, receivers)
        e, m = _tc_edge(a_s, a_r, e, env, onesd, m8, w1cg, b1g,
                        W2[t], b2[t].reshape(1, D))
        parts = _sc_scatter(m, recv2d, zeros_nd)
        h = _tc_node(h, parts[0], parts[1],
                     U1[t][:D], U1[t][D:], c1[t].reshape(1, H),
                     U2[t], c2[t].reshape(1, D))

    Wh2p = jnp.zeros((H, D), f32).at[:, :3].set(Wh2)
    bh2p = jnp.zeros((1, D), f32).at[0, :3].set(bh2)
    pred = _tc_head(h, Wh1, bh1.reshape(1, H), Wh2p, bh2p)
    return pred[:, :3]


# speedup vs baseline: 1.0177x; 1.0177x over previous
import functools

import jax
import jax.numpy as jnp
from jax import lax
from jax.experimental import pallas as pl
from jax.experimental.pallas import tpu as pltpu
from jax.experimental.pallas import tpu_sc as plsc

N = 10000
E = 320000
D = 128
H = 128
T = 3
NUM_BASES = 8
SH_DIM = 9
BOND_DIM = 16
R_MAX = 5.0

NC = 2
NS = 16
NW = NC * NS

CH = 80
KBUF = 5
EPW = E // NW
NPAD = 10240

_mesh = plsc.VectorSubcoreMesh(core_axis_name="c", subcore_axis_name="s",
                               num_cores=NC, num_subcores=NS)


def _wid():
    return lax.axis_index("c") * NS + lax.axis_index("s")


def _gather_small(tbl_hbm, idx_v, out_hbm, rows, sems, bpw, base):
    nchunks = bpw // CH
    descs = []
    for c in range(nchunks):
        descs.append(pltpu.async_copy(
            tbl_hbm.at[idx_v.at[pl.ds(c * CH, CH)]], rows[c], sems[c]))
    for c in range(nchunks):
        descs[c].wait()
        pltpu.sync_copy(rows[c], out_hbm.at[pl.ds(base + c * CH, CH), :])


GCH = 128
GK = 6
_GNFULL = EPW // GCH
_GTAIL = EPW - _GNFULL * GCH


def _gather_job(tbl_hbm, idx_v, out_hbm, rows, gsems, osems, base):
    def _wait_out(b):
        pltpu.make_async_copy(
            rows[b], out_hbm.at[pl.ds(base, GCH), :], osems[b]).wait()

    @pl.loop(0, _GNFULL // GK)
    def _group(g):
        c0 = g * GK
        descs = []
        for b in range(GK):
            @pl.when(g > 0)
            def _w(b=b):
                _wait_out(b)
            descs.append(pltpu.async_copy(
                tbl_hbm.at[idx_v.at[pl.ds((c0 + b) * GCH, GCH)]],
                rows[b], gsems[b]))
        for b in range(GK):
            descs[b].wait()
            pltpu.async_copy(
                rows[b], out_hbm.at[pl.ds(base + (c0 + b) * GCH, GCH), :],
                osems[b])

    for b in range(GK):
        _wait_out(b)
    toff = _GNFULL * GCH
    pltpu.async_copy(
        tbl_hbm.at[idx_v.at[pl.ds(toff, _GTAIL)]],
        rows[0].at[pl.ds(0, _GTAIL), :], gsems[0]).wait()
    pltpu.sync_copy(rows[0].at[pl.ds(0, _GTAIL), :],
                    out_hbm.at[pl.ds(base + toff, _GTAIL), :])


def _sc_gatherpos_body(pos_hbm, send_hbm, recv_hbm,
                       ps_hbm, pr_hbm, idx_v, *bufs):
    wid = _wid()
    base = wid * EPW
    rows = list(bufs[:GK])
    gsems = list(bufs[GK:2 * GK])
    osems = list(bufs[2 * GK:3 * GK])
    for idx_hbm, tbl, out in ((send_hbm, pos_hbm, ps_hbm),
                              (recv_hbm, pos_hbm, pr_hbm)):
        pltpu.sync_copy(idx_hbm.at[pl.ds(base, EPW)], idx_v)
        _gather_job(tbl, idx_v, out, rows, gsems, osems, base)


_sc_gatherpos = pl.kernel(
    _sc_gatherpos_body,
    out_type=[jax.ShapeDtypeStruct((E, 16), jnp.float32),
              jax.ShapeDtypeStruct((E, 16), jnp.float32)],
    mesh=_mesh,
    compiler_params=pltpu.CompilerParams(use_tc_tiling_on_sc=False),
    scratch_types=[pltpu.VMEM((EPW,), jnp.int32)]
    + [pltpu.VMEM((GCH, 16), jnp.float32) for _ in range(GK)]
    + [pltpu.SemaphoreType.DMA for _ in range(2 * GK)],
)


def _sc_gather_atoms_body(tbl_hbm, idx_hbm, out_hbm,
                          idx_v, r0, r1, r2, r3, s0, s1, s2, s3):
    wid = _wid()
    bpw = NPAD // NW
    base = wid * bpw
    pltpu.sync_copy(idx_hbm.at[pl.ds(base, bpw)], idx_v)
    _gather_small(tbl_hbm, idx_v, out_hbm, [r0, r1, r2, r3],
                  [s0, s1, s2, s3], bpw, base)


_sc_gather_atoms = pl.kernel(
    _sc_gather_atoms_body,
    out_type=jax.ShapeDtypeStruct((NPAD, D), jnp.float32),
    mesh=_mesh,
    compiler_params=pltpu.CompilerParams(use_tc_tiling_on_sc=False),
    scratch_types=[pltpu.VMEM((NPAD // NW,), jnp.int32)]
    + [pltpu.VMEM((CH, D), jnp.float32) for _ in range(4)]
    + [pltpu.SemaphoreType.DMA for _ in range(4)],
)


def _sc_gather2_body(ts_hbm, send_hbm, tr_hbm, recv_hbm,
                     as_hbm, ar_hbm, idx_v, *bufs):
    wid = _wid()
    base = wid * EPW
    rows = list(bufs[:GK])
    gsems = list(bufs[GK:2 * GK])
    osems = list(bufs[2 * GK:3 * GK])
    for idx_hbm, tbl, out in ((send_hbm, ts_hbm, as_hbm),
                              (recv_hbm, tr_hbm, ar_hbm)):
        pltpu.sync_copy(idx_hbm.at[pl.ds(base, EPW)], idx_v)
        _gather_job(tbl, idx_v, out, rows, gsems, osems, base)


_sc_gather2 = pl.kernel(
    _sc_gather2_body,
    out_type=[jax.ShapeDtypeStruct((E, 72), jnp.float32),
              jax.ShapeDtypeStruct((E, 72), jnp.float32)],
    mesh=_mesh,
    compiler_params=pltpu.CompilerParams(use_tc_tiling_on_sc=False),
    scratch_types=[pltpu.VMEM((EPW,), jnp.int32)]
    + [pltpu.VMEM((GCH, 72), jnp.float32) for _ in range(GK)]
    + [pltpu.SemaphoreType.DMA for _ in range(2 * GK)],
)


_NCHUNK = EPW // CH
_ROWS_PER_TILE = N // NS


def _sc_scatter_body(m_hbm, recv2d_hbm, zeros_hbm, out_hbm,
                     idxs_v, mb0, mb1, acc_sh, lm0, lm1, ss0, ss1):
    cid = lax.axis_index("c")
    sid = lax.axis_index("s")
    wid = cid * NS + sid
    base = wid * EPW
    pltpu.sync_copy(zeros_hbm.at[pl.ds(sid * _ROWS_PER_TILE, _ROWS_PER_TILE), :],
                    acc_sh.at[pl.ds(sid * _ROWS_PER_TILE, _ROWS_PER_TILE), :])
    pltpu.sync_copy(recv2d_hbm.at[pl.ds(wid * _NCHUNK, _NCHUNK), :], idxs_v)
    plsc.subcore_barrier()

    mbufs = (mb0, mb1)
    lsems = (lm0, lm1)
    ssems = (ss0, ss1)

    @pl.loop(0, _NCHUNK // 2)
    def _group(g):
        loads = []
        for b in range(2):
            loads.append(pltpu.async_copy(
                m_hbm.at[pl.ds(base + (g * 2 + b) * CH, CH), :],
                mbufs[b], lsems[b]))
        scats = []
        for b in range(2):
            loads[b].wait()
            scats.append(pltpu.async_copy(
                mbufs[b], acc_sh.at[idxs_v.at[g * 2 + b]],
                ssems[b], add=True))
        for b in range(2):
            scats[b].wait()

    c = _NCHUNK - 1
    pltpu.async_copy(m_hbm.at[pl.ds(base + c * CH, CH), :], mb0, lm0).wait()
    pltpu.async_copy(mb0, acc_sh.at[idxs_v.at[c]], ss0, add=True).wait()

    plsc.subcore_barrier()
    pltpu.sync_copy(acc_sh.at[pl.ds(sid * _ROWS_PER_TILE, _ROWS_PER_TILE), :],
                    out_hbm.at[cid, pl.ds(sid * _ROWS_PER_TILE, _ROWS_PER_TILE), :])


_sc_scatter = pl.kernel(
    _sc_scatter_body,
    out_type=jax.ShapeDtypeStruct((NC, N, D), jnp.float32),
    mesh=_mesh,
    compiler_params=pltpu.CompilerParams(use_tc_tiling_on_sc=False),
    scratch_types=[pltpu.VMEM((EPW // CH, CH), jnp.int32),
                   pltpu.VMEM((CH, D), jnp.float32),
                   pltpu.VMEM((CH, D), jnp.float32),
                   pltpu.VMEM_SHARED((N, D), jnp.float32)]
    + [pltpu.SemaphoreType.DMA for _ in range(4)],
)


BE = 2000
BN = 2000

_S3 = 3.0 ** 0.5
_S5 = 5.0 ** 0.5
_S15 = 15.0 ** 0.5
_PI = 3.141592653589793


def _rms(x):
    return x * lax.rsqrt(jnp.mean(x * x, axis=-1, keepdims=True) + 1e-6)


def _silu(x):
    return x * jax.nn.sigmoid(x)


def _geom_body(ps_ref, pr_ref, oh_ref, ones_ref, mxyz_ref, mb8_ref,
               wp_ref, wbond_ref, b0_ref, e0_ref, env_ref):
    f32 = jnp.float32
    vec = ps_ref[:, :] - pr_ref[:, :]
    d2 = jnp.dot(vec * vec, ones_ref[:, :],
                 preferred_element_type=f32) + 1e-12
    inv_d = lax.rsqrt(d2)
    d = d2 * inv_d
    inv_dd = 1.0 / (d + 1e-9)
    xyz = jnp.dot(vec, mxyz_ref[:, :], preferred_element_type=f32)
    xu = xyz[:, 0:16] * inv_d
    yu = xyz[:, 16:32] * inv_d
    zu = xyz[:, 32:48] * inv_d
    lane = lax.broadcasted_iota(jnp.int32, (BE, 16), 1)
    coef = jnp.where(lane < 8, (lane + 1).astype(f32),
                     jnp.where(lane == 8, 1.0, 0.0))
    off = jnp.where(lane == 8, _PI / 2.0, 0.0)
    s = jnp.sin(d * (coef * (_PI / R_MAX)) + off)
    s8 = jnp.dot(s, mb8_ref[:, :], preferred_element_type=f32)
    env = jnp.where(d < R_MAX, 0.5 * (s8 + 1.0), 0.0)
    g = (s * inv_dd) * (((2.0 / R_MAX) ** 0.5)) * env
    sh_rest = [_S3 * xu, _S3 * yu, _S3 * zu,
               _S15 * xu * yu, _S15 * yu * zu,
               (_S5 / 2.0) * (3.0 * zu * zu - 1.0),
               _S15 * xu * zu, (_S15 / 2.0) * (xu * xu - yu * yu)]
    outer = jnp.concatenate(
        [g] + [g * sj for sj in sh_rest], axis=1)
    e0 = (jnp.dot(outer, wp_ref[:, :], preferred_element_type=f32)
          + jnp.dot(oh_ref[:, :], wbond_ref[:, :], preferred_element_type=f32)
          + b0_ref[:, :])
    e0_ref[:, :] = e0
    env_ref[:, :] = env[:, 0:8]


def _full(shape):
    return pl.BlockSpec(shape, lambda i: (0,) * len(shape))


_tc_geom = pl.pallas_call(
    _geom_body,
    grid=(E // BE,),
    in_specs=[pl.BlockSpec((BE, 16), lambda i: (i, 0)),
              pl.BlockSpec((BE, 16), lambda i: (i, 0)),
              pl.BlockSpec((BE, 8), lambda i: (i, 0)),
              _full((16, 16)), _full((16, 48)), _full((16, 16)),
              _full((144, D)), _full((8, D)), _full((1, D))],
    out_specs=[pl.BlockSpec((BE, D), lambda i: (i, 0)),
               pl.BlockSpec((BE, 8), lambda i: (i, 0))],
    out_shape=[jax.ShapeDtypeStruct((E, D), jnp.float32),
               jax.ShapeDtypeStruct((E, 8), jnp.float32)],
    compiler_params=pltpu.CompilerParams(
        dimension_semantics=("arbitrary",)),
)


_HIMASK = -65536


def _pack2(x, k=72):
    hi = lax.bitcast_convert_type(x[:, :k], jnp.int32)
    lo = lax.bitcast_convert_type(x[:, k:], jnp.int32)
    hi = (hi + 0x8000) & _HIMASK
    lo = lax.shift_right_logical(lo + 0x8000, 16)
    return lax.bitcast_convert_type(hi | lo, jnp.float32)


def _unpack2(p):
    u = lax.bitcast_convert_type(p, jnp.int32)
    hi = lax.bitcast_convert_type(u & _HIMASK, jnp.float32)
    lo = lax.bitcast_convert_type(lax.shift_left(u, 16), jnp.float32)
    return jnp.concatenate([hi, lo], axis=1)


def _proj_body(h_ref, wsa_ref, wsb_ref, ts_ref, tr_ref):
    h = h_ref[:, :]
    ts_ref[:, :] = _pack2(
        jnp.dot(h, wsa_ref[:, :], preferred_element_type=jnp.float32))
    tr_ref[:, :] = _pack2(
        jnp.dot(h, wsb_ref[:, :], preferred_element_type=jnp.float32))


_tc_proj = pl.pallas_call(
    _proj_body,
    grid=(N // BN,),
    in_specs=[pl.BlockSpec((BN, D), lambda i: (i, 0)),
              _full((D, 144)), _full((D, 144))],
    out_specs=[pl.BlockSpec((BN, 72), lambda i: (i, 0)),
               pl.BlockSpec((BN, 72), lambda i: (i, 0))],
    out_shape=[jax.ShapeDtypeStruct((N, 72), jnp.float32),
               jax.ShapeDtypeStruct((N, 72), jnp.float32)],
    compiler_params=pltpu.CompilerParams(
        dimension_semantics=("arbitrary",)),
)


def _edge_body(as_ref, ar_ref, e_ref, env_ref, onesd_ref, m8_ref,
               w1cg_ref, b1g_ref, w2_ref, b2_ref, enew_ref, m_ref):
    f32 = jnp.float32
    e = e_ref[:, :]
    tmp = (_unpack2(as_ref[:, :]) + _unpack2(ar_ref[:, :])
           + jnp.dot(e, w1cg_ref[:, :], preferred_element_type=f32)
           + b1g_ref[:, :])
    pre = tmp[:, :D]
    msq = jnp.dot(pre * pre, onesd_ref[:, :],
                  preferred_element_type=f32) * (1.0 / D)
    nrm = pre * lax.rsqrt(msq + 1e-6)
    act = _silu(nrm)
    m2 = jnp.dot(act, w2_ref[:, :], preferred_element_type=f32) + b2_ref[:, :]
    gl8 = tmp[:, D:D + 8]
    gev8 = jax.nn.sigmoid(gl8) * env_ref[:, :]
    m = m2 * jnp.dot(gev8, m8_ref[:, :], preferred_element_type=f32)
    enew_ref[:, :] = e + m
    m_ref[:, :] = m


_tc_edge = pl.pallas_call(
    _edge_body,
    grid=(E // BE,),
    in_specs=[pl.BlockSpec((BE, 72), lambda i: (i, 0)),
              pl.BlockSpec((BE, 72), lambda i: (i, 0)),
              pl.BlockSpec((BE, D), lambda i: (i, 0)),
              pl.BlockSpec((BE, 8), lambda i: (i, 0)),
              _full((D, D)), _full((8, D)),
              _full((D, 144)), _full((1, 144)),
              _full((H, D)), _full((1, D))],
    out_specs=[pl.BlockSpec((BE, D), lambda i: (i, 0)),
               pl.BlockSpec((BE, D), lambda i: (i, 0))],
    out_shape=[jax.ShapeDtypeStruct((E, D), jnp.float32),
               jax.ShapeDtypeStruct((E, D), jnp.float32)],
    compiler_params=pltpu.CompilerParams(
        dimension_semantics=("arbitrary",)),
)


def _node_body(h_ref, p0_ref, p1_ref, u1a_ref, u1b_ref, c1_ref,
               u2_ref, c2_ref, hnew_ref):
    h = h_ref[:, :]
    agg = p0_ref[:, :] + p1_ref[:, :]
    pre = (jnp.dot(h, u1a_ref[:, :], preferred_element_type=jnp.float32)
           + jnp.dot(agg, u1b_ref[:, :], preferred_element_type=jnp.float32)
           + c1_ref[:, :])
    hnew_ref[:, :] = h + jnp.dot(_silu(_rms(pre)), u2_ref[:, :],
                                 preferred_element_type=jnp.float32) + c2_ref[:, :]


_tc_node = pl.pallas_call(
    _node_body,
    grid=(N // BN,),
    in_specs=[pl.BlockSpec((BN, D), lambda i: (i, 0)),
              pl.BlockSpec((BN, D), lambda i: (i, 0)),
              pl.BlockSpec((BN, D), lambda i: (i, 0)),
              _full((D, H)), _full((D, H)), _full((1, H)),
              _full((H, D)), _full((1, D))],
    out_specs=pl.BlockSpec((BN, D), lambda i: (i, 0)),
    out_shape=jax.ShapeDtypeStruct((N, D), jnp.float32),
    compiler_params=pltpu.CompilerParams(
        dimension_semantics=("arbitrary",)),
)


def _head_body(h_ref, w1_ref, b1_ref, w2_ref, b2_ref, out_ref):
    pre = jnp.dot(h_ref[:, :], w1_ref[:, :],
                  preferred_element_type=jnp.float32) + b1_ref[:, :]
    out_ref[:, :] = jnp.dot(_silu(_rms(pre)), w2_ref[:, :],
                            preferred_element_type=jnp.float32) + b2_ref[:, :]


_tc_head = pl.pallas_call(
    _head_body,
    grid=(N // BN,),
    in_specs=[pl.BlockSpec((BN, D), lambda i: (i, 0)),
              _full((D, H)), _full((1, H)),
              _full((H, D)), _full((1, D))],
    out_specs=pl.BlockSpec((BN, D), lambda i: (i, 0)),
    out_shape=jax.ShapeDtypeStruct((N, D), jnp.float32),
    compiler_params=pltpu.CompilerParams(
        dimension_semantics=("arbitrary",)),
)


def kernel(pos, atomic_numbers, senders, receivers, bond_types, batch,
           num_graphs, c_noise, c_in, atom_table, bond_table, W_e0, b_e0,
           W1, b1, W2, b2, Wg, bg, U1, c1, U2, c2, Wh1, bh1, Wh2, bh2):
    del batch, num_graphs, c_noise
    f32 = jnp.float32
    i32 = jnp.int32
    senders = senders.astype(i32)
    receivers = receivers.astype(i32)
    bond_types = bond_types.astype(i32)
    atomic_numbers = atomic_numbers.astype(i32)

    unscaled = pos.astype(f32) / c_in.astype(f32)
    pos_pad = jnp.zeros((N, 16), f32).at[:, :3].set(unscaled)
    at_pad = jnp.zeros((NPAD,), i32).at[:N].set(atomic_numbers)
    recv2d = receivers.reshape(E // CH, CH)
    zeros_nd = jnp.zeros((N, D), f32)

    Wp = W_e0[:NUM_BASES * SH_DIM].reshape(NUM_BASES, SH_DIM, D)
    Wp = Wp.transpose(1, 0, 2)
    Wp = jnp.concatenate(
        [Wp, jnp.zeros((SH_DIM, 16 - NUM_BASES, D), f32)], axis=1
    ).reshape(SH_DIM * 16, D)
    Wbond = jnp.zeros((8, BOND_DIM), f32).at[:5].set(
        bond_table.astype(f32)) @ W_e0[NUM_BASES * SH_DIM:]
    onehot = (bond_types[:, None] == jnp.arange(8, dtype=i32)[None, :]
              ).astype(f32)
    b_e0r = b_e0.reshape(1, D)

    ones16 = jnp.ones((16, 16), f32)
    eye16 = jnp.eye(16, dtype=f32)
    mxyz = jnp.concatenate(
        [jnp.outer(eye16[0], jnp.ones(16, f32)),
         jnp.outer(eye16[1], jnp.ones(16, f32)),
         jnp.outer(eye16[2], jnp.ones(16, f32))], axis=1)
    mb8 = jnp.outer(eye16[8], jnp.ones(16, f32))
    onesd = jnp.ones((D, D), f32)
    m8 = jnp.outer(jnp.eye(8, dtype=f32)[0], jnp.ones(D, f32))

    ps, pr = _sc_gatherpos(pos_pad, senders, receivers)
    h_full = _sc_gather_atoms(atom_table.astype(f32), at_pad)
    h = h_full[:N]

    e, env = _tc_geom(ps, pr, onehot, ones16, mxyz, mb8, Wp, Wbond, b_e0r)

    for t in range(T):
        W1t = W1[t]
        Wgt = Wg[t]
        wsa = jnp.concatenate(
            [W1t[:D], jnp.zeros((D, 16), f32).at[:, 0:1].set(Wgt[:D])], axis=1)
        wsb = jnp.concatenate(
            [W1t[D:2 * D], jnp.zeros((D, 16), f32).at[:, 0:1].set(Wgt[D:2 * D])],
            axis=1)
        w1cg = jnp.concatenate(
            [W1t[2 * D:], jnp.zeros((D, 16), f32).at[:, 0:1].set(Wgt[2 * D:])],
            axis=1)
        b1g = jnp.zeros((1, 144), f32).at[0, :D].set(b1[t]).at[0, D].set(bg[t, 0])

        ts, tr = _tc_proj(h, wsa, wsb)
        a_s, a_r = _sc_gather2(ts, senders, tr, receivers)
        e, m = _tc_edge(a_s, a_r, e, env, onesd, m8, w1cg, b1g,
                        W2[t], b2[t].reshape(1, D))
        parts = _sc_scatter(m, recv2d, zeros_nd)
        h = _tc_node(h, parts[0], parts[1],
                     U1[t][:D], U1[t][D:], c1[t].reshape(1, H),
                     U2[t], c2[t].reshape(1, D))

    Wh2p = jnp.zeros((H, D), f32).at[:, :3].set(Wh2)
    bh2p = jnp.zeros((1, D), f32).at[0, :3].set(bh2)
    pred = _tc_head(h, Wh1, bh1.reshape(1, H), Wh2p, bh2p)
    return pred[:, :3]
